# baseline (speedup 1.0000x reference)
import jax
import jax.numpy as jnp
from jax.experimental import pallas as pl
from jax.experimental.pallas import tpu as pltpu

_G = 32
_RP = 512
_NA = 2


def _mlp_kernel(x3_ref, w1_ref, b1_ref, w2_ref, b2_ref, w3_ref, b3_ref,
                o_ref):
    rp = x3_ref.shape[0]
    x = x3_ref[...].reshape(2 * rp, _G * 4)
    h1 = jnp.dot(x, w1_ref[...], preferred_element_type=jnp.float32)
    h1 = jnp.maximum(h1 + b1_ref[...], 0.0)
    w2 = w2_ref[...]
    b2 = b2_ref[...]
    chunks = []
    for m in range(h1.shape[1] // 256):
        cseg = jnp.dot(h1[:, 256 * m:256 * (m + 1)], w2,
                       preferred_element_type=jnp.float32)
        chunks.append(jnp.maximum(cseg + b2, 0.0))
    h2 = jnp.concatenate(chunks, axis=1)
    q = jnp.dot(h2, w3_ref[...], preferred_element_type=jnp.float32)
    q = q + b3_ref[...]
    q3 = q.reshape(rp, 2, q.shape[1])
    o_ref[...] = jnp.concatenate([q3[:, 0, :], q3[:, 1, :]], axis=1)


def kernel(x, w1, b1, w2, b2, w3, b3):
    B, S = x.shape
    gr2 = B // (2 * _G)
    rp = _RP if gr2 % _RP == 0 else gr2
    x3 = x.reshape(gr2, 2, _G * S)

    eye_g = jnp.eye(_G, dtype=jnp.float32)
    eye_8 = jnp.eye(8, dtype=jnp.float32)
    w1b = jnp.kron(eye_g, w1)
    b1b = jnp.tile(b1, (1, _G))
    w2b = jnp.kron(eye_8, w2)
    b2b = jnp.tile(b2, (1, 8))
    w3b = jnp.kron(eye_g, w3[:, :_NA])
    b3b = jnp.tile(b3[:, :_NA], (1, _G))

    fixed = lambda i: (0, 0)
    q = pl.pallas_call(
        _mlp_kernel,
        out_shape=jax.ShapeDtypeStruct((gr2, 2 * _NA * _G), jnp.float32),
        grid=(gr2 // rp,),
        in_specs=[
            pl.BlockSpec((rp, 2, _G * S), lambda i: (i, 0, 0)),
            pl.BlockSpec(w1b.shape, fixed), pl.BlockSpec(b1b.shape, fixed),
            pl.BlockSpec(w2b.shape, fixed), pl.BlockSpec(b2b.shape, fixed),
            pl.BlockSpec(w3b.shape, fixed), pl.BlockSpec(b3b.shape, fixed),
        ],
        out_specs=pl.BlockSpec((rp, 2 * _NA * _G), lambda i: (i, 0)),
        compiler_params=pltpu.CompilerParams(
            dimension_semantics=("parallel",)),
    )(x3, w1b, b1b, w2b, b2b, w3b, b3b)
    return q.reshape(B, _NA)


# speedup vs baseline: 4.8284x; 4.8284x over previous
import jax
import jax.numpy as jnp
from jax.experimental import pallas as pl
from jax.experimental.pallas import tpu as pltpu

_R = 16384
_NQ = 8
_NA = 2

_CN = (((0,), (1,)), ((), ()))
_CC = (((0,), (0,)), ((), ()))


def _start_block(x_hbm, xv, isem, i, slot, r):
    sub = r // _NQ
    for k in range(_NQ):
        pltpu.make_async_copy(
            x_hbm.at[pl.ds(i * r + k * sub, sub), :],
            xv.at[slot, pl.ds(k * sub, sub), :],
            isem.at[slot, k],
        ).start()


def _wait_block(x_hbm, xv, isem, i, slot, r):
    sub = r // _NQ
    for k in range(_NQ):
        pltpu.make_async_copy(
            x_hbm.at[pl.ds(i * r + k * sub, sub), :],
            xv.at[slot, pl.ds(k * sub, sub), :],
            isem.at[slot, k],
        ).wait()


def _make_kernel(r, nsteps):
    def _kern(x_hbm, w1_ref, b1t_ref, w2_ref, b2t_ref, w3a_ref, o_ref,
              xv, isem):
        c = pl.program_id(0)
        j = pl.program_id(1)
        i = c * nsteps + j
        slot = jax.lax.rem(j, 2)

        @pl.when(j == 0)
        def _prologue():
            _start_block(x_hbm, xv, isem, i, 0, r)

        _wait_block(x_hbm, xv, isem, i, slot, r)

        @pl.when(j + 1 < nsteps)
        def _prefetch():
            _start_block(x_hbm, xv, isem, i + 1, jax.lax.rem(j + 1, 2), r)

        x = xv[slot]
        h1 = jax.lax.dot_general(w1_ref[...], x, _CN,
                                 preferred_element_type=jnp.float32)
        h1 = jnp.maximum(h1 + b1t_ref[...], 0.0)
        h2 = jax.lax.dot_general(w2_ref[...], h1, _CC,
                                 preferred_element_type=jnp.float32)
        h2 = jnp.maximum(h2 + b2t_ref[...], 0.0)
        ones = jnp.ones((1, h2.shape[1]), jnp.float32)
        h2a = jnp.concatenate([h2, ones], axis=0)
        q = jax.lax.dot_general(w3a_ref[...], h2a, _CC,
                                preferred_element_type=jnp.float32)
        o_ref[...] = q

    return _kern


def kernel(x, w1, b1, w2, b2, w3, b3):
    B, S = x.shape
    r = _R if B % (2 * _R) == 0 else B
    nsteps = B // r // 2 if r != B else 1
    ncores = 2 if r != B else 1

    b1t = b1.T
    b2t = b2.T
    w3a = jnp.concatenate([w3[:, :_NA], b3[:, :_NA]], axis=0)

    fixed = lambda c, j: (0, 0)
    qt = pl.pallas_call(
        _make_kernel(r, nsteps),
        out_shape=jax.ShapeDtypeStruct((_NA, B), jnp.float32),
        grid=(ncores, nsteps),
        in_specs=[
            pl.BlockSpec(memory_space=pltpu.MemorySpace.HBM),
            pl.BlockSpec(w1.shape, fixed), pl.BlockSpec(b1t.shape, fixed),
            pl.BlockSpec(w2.shape, fixed), pl.BlockSpec(b2t.shape, fixed),
            pl.BlockSpec(w3a.shape, fixed),
        ],
        out_specs=pl.BlockSpec((_NA, r), lambda c, j: (0, c * nsteps + j)),
        scratch_shapes=[
            pltpu.VMEM((2, r, 4), jnp.float32),
            pltpu.SemaphoreType.DMA((2, _NQ)),
        ],
        compiler_params=pltpu.CompilerParams(
            dimension_semantics=("parallel", "arbitrary")),
    )(x, w1, b1t, w2, b2t, w3a)
    return qt.T


# speedup vs baseline: 37.6385x; 7.7952x over previous
import jax
import jax.numpy as jnp
from jax.experimental import pallas as pl
from jax.experimental.pallas import tpu as pltpu

_R = 32768
_NA = 2

_CC = (((0,), (0,)), ((), ()))


def _mlp_kernel(xt_ref, w1_ref, b1t_ref, w2_ref, b2t_ref, w3a_ref, o_ref):
    xt = xt_ref[...]
    h1 = jax.lax.dot_general(w1_ref[...], xt, _CC,
                             preferred_element_type=jnp.float32)
    h1 = jnp.maximum(h1 + b1t_ref[...], 0.0)
    h2 = jax.lax.dot_general(w2_ref[...], h1, _CC,
                             preferred_element_type=jnp.float32)
    h2 = jnp.maximum(h2 + b2t_ref[...], 0.0)
    ones = jnp.ones((1, h2.shape[1]), jnp.float32)
    h2a = jnp.concatenate([h2, ones], axis=0)
    q = jax.lax.dot_general(w3a_ref[...], h2a, _CC,
                            preferred_element_type=jnp.float32)
    o_ref[...] = q


def kernel(x, w1, b1, w2, b2, w3, b3):
    B, S = x.shape
    r = _R if B % _R == 0 else B

    xt = x.T
    b1t = b1.T
    b2t = b2.T
    w3a = jnp.concatenate([w3[:, :_NA], b3[:, :_NA]], axis=0)

    fixed = lambda i: (0, 0)
    qt = pl.pallas_call(
        _mlp_kernel,
        out_shape=jax.ShapeDtypeStruct((_NA, B), jnp.float32),
        grid=(B // r,),
        in_specs=[
            pl.BlockSpec((S, r), lambda i: (0, i)),
            pl.BlockSpec(w1.shape, fixed), pl.BlockSpec(b1t.shape, fixed),
            pl.BlockSpec(w2.shape, fixed), pl.BlockSpec(b2t.shape, fixed),
            pl.BlockSpec(w3a.shape, fixed),
        ],
        out_specs=pl.BlockSpec((_NA, r), lambda i: (0, i)),
        compiler_params=pltpu.CompilerParams(
            dimension_semantics=("parallel",)),
    )(xt, w1, b1t, w2, b2t, w3a)
    return qt.T


# speedup vs baseline: 39.3025x; 1.0442x over previous
import jax
import jax.numpy as jnp
from jax.experimental import pallas as pl
from jax.experimental.pallas import tpu as pltpu

_R = 32768
_NA = 2

_CC = (((0,), (0,)), ((), ()))


def _mlp_kernel(xt_ref, w1_ref, b1t_ref, w2_ref, b2t_ref, w3t_ref, b3t_ref,
                o_ref):
    xt = xt_ref[...]
    h1 = jax.lax.dot_general(w1_ref[...], xt, _CC,
                             preferred_element_type=jnp.float32)
    h1 = jnp.maximum(h1 + b1t_ref[...], 0.0)
    h2 = jax.lax.dot_general(w2_ref[...], h1, _CC,
                             preferred_element_type=jnp.float32)
    h2 = jnp.maximum(h2 + b2t_ref[...], 0.0)
    q = jax.lax.dot_general(w3t_ref[...], h2, _CC,
                            preferred_element_type=jnp.float32)
    o_ref[...] = q + b3t_ref[...]


def kernel(x, w1, b1, w2, b2, w3, b3):
    B, S = x.shape
    r = _R if B % _R == 0 else B

    xt = x.T
    b1t = b1.T
    b2t = b2.T
    w3s = w3[:, :_NA]
    b3t = b3[:, :_NA].T

    fixed = lambda i: (0, 0)
    qt = pl.pallas_call(
        _mlp_kernel,
        out_shape=jax.ShapeDtypeStruct((_NA, B), jnp.float32),
        grid=(B // r,),
        in_specs=[
            pl.BlockSpec((S, r), lambda i: (0, i)),
            pl.BlockSpec(w1.shape, fixed), pl.BlockSpec(b1t.shape, fixed),
            pl.BlockSpec(w2.shape, fixed), pl.BlockSpec(b2t.shape, fixed),
            pl.BlockSpec(w3s.shape, fixed), pl.BlockSpec(b3t.shape, fixed),
        ],
        out_specs=pl.BlockSpec((_NA, r), lambda i: (0, i)),
        compiler_params=pltpu.CompilerParams(
            dimension_semantics=("parallel",)),
    )(xt, w1, b1t, w2, b2t, w3s, b3t)
    return qt.T


# speedup vs baseline: 42.0166x; 1.0691x over previous
import jax
import jax.numpy as jnp
from jax.experimental import pallas as pl
from jax.experimental.pallas import tpu as pltpu

_R = 65536
_NA = 2

_CC = (((0,), (0,)), ((), ()))


def _mlp_kernel(xt_ref, w1_ref, b1t_ref, w2_ref, b2t_ref, w3t_ref, b3t_ref,
                o_ref):
    xt = xt_ref[...]
    h1 = jax.lax.dot_general(w1_ref[...], xt, _CC,
                             preferred_element_type=jnp.float32)
    h1 = jnp.maximum(h1 + b1t_ref[...], 0.0)
    h2 = jax.lax.dot_general(w2_ref[...], h1, _CC,
                             preferred_element_type=jnp.float32)
    h2 = jnp.maximum(h2 + b2t_ref[...], 0.0)
    q = jax.lax.dot_general(w3t_ref[...], h2, _CC,
                            preferred_element_type=jnp.float32)
    o_ref[...] = q + b3t_ref[...]


def kernel(x, w1, b1, w2, b2, w3, b3):
    B, S = x.shape
    r = _R if B % _R == 0 else B

    xt = x.T
    b1t = b1.T
    b2t = b2.T
    w3s = w3[:, :_NA]
    b3t = b3[:, :_NA].T

    fixed = lambda i: (0, 0)
    qt = pl.pallas_call(
        _mlp_kernel,
        out_shape=jax.ShapeDtypeStruct((_NA, B), jnp.float32),
        grid=(B // r,),
        in_specs=[
            pl.BlockSpec((S, r), lambda i: (0, i)),
            pl.BlockSpec(w1.shape, fixed), pl.BlockSpec(b1t.shape, fixed),
            pl.BlockSpec(w2.shape, fixed), pl.BlockSpec(b2t.shape, fixed),
            pl.BlockSpec(w3s.shape, fixed), pl.BlockSpec(b3t.shape, fixed),
        ],
        out_specs=pl.BlockSpec((_NA, r), lambda i: (0, i)),
        compiler_params=pltpu.CompilerParams(
            dimension_semantics=("parallel",)),
    )(xt, w1, b1t, w2, b2t, w3s, b3t)
    return qt.T


# speedup vs baseline: 43.2971x; 1.0305x over previous
import jax
import jax.numpy as jnp
from jax.experimental import pallas as pl
from jax.experimental.pallas import tpu as pltpu

_R = 131072
_NA = 2

_CC = (((0,), (0,)), ((), ()))


def _mlp_kernel(xt_ref, w1_ref, b1t_ref, w2_ref, b2t_ref, w3t_ref, b3t_ref,
                o_ref):
    xt = xt_ref[...]
    h1 = jax.lax.dot_general(w1_ref[...], xt, _CC,
                             preferred_element_type=jnp.float32)
    h1 = jnp.maximum(h1 + b1t_ref[...], 0.0)
    h2 = jax.lax.dot_general(w2_ref[...], h1, _CC,
                             preferred_element_type=jnp.float32)
    h2 = jnp.maximum(h2 + b2t_ref[...], 0.0)
    q = jax.lax.dot_general(w3t_ref[...], h2, _CC,
                            preferred_element_type=jnp.float32)
    o_ref[...] = q + b3t_ref[...]


def kernel(x, w1, b1, w2, b2, w3, b3):
    B, S = x.shape
    r = _R if B % _R == 0 else B

    xt = x.T
    b1t = b1.T
    b2t = b2.T
    w3s = w3[:, :_NA]
    b3t = b3[:, :_NA].T

    fixed = lambda i: (0, 0)
    qt = pl.pallas_call(
        _mlp_kernel,
        out_shape=jax.ShapeDtypeStruct((_NA, B), jnp.float32),
        grid=(B // r,),
        in_specs=[
            pl.BlockSpec((S, r), lambda i: (0, i)),
            pl.BlockSpec(w1.shape, fixed), pl.BlockSpec(b1t.shape, fixed),
            pl.BlockSpec(w2.shape, fixed), pl.BlockSpec(b2t.shape, fixed),
            pl.BlockSpec(w3s.shape, fixed), pl.BlockSpec(b3t.shape, fixed),
        ],
        out_specs=pl.BlockSpec((_NA, r), lambda i: (0, i)),
        compiler_params=pltpu.CompilerParams(
            dimension_semantics=("parallel",)),
    )(xt, w1, b1t, w2, b2t, w3s, b3t)
    return qt.T
